# Initial kernel scaffold; baseline (speedup 1.0000x reference)
import jax
import jax.numpy as jnp
from jax.experimental import pallas as pl


def kernel(x, emb, W1, b1, W2, b2):
    raise NotImplementedError("write your pallas kernel here")


# speedup vs baseline: 24.9960x; 24.9960x over previous
import functools

import jax
import jax.numpy as jnp
from jax import lax
from jax.experimental import pallas as pl
from jax.experimental.pallas import tpu as pltpu
from jax.experimental.pallas import tpu_sc as plsc

VOCAB = 100000
EMBED = 16
BATCH = 4096
HIST = 200

NC = 2
NS = 16
NW = NC * NS
ROWS_PER_W = BATCH // NW
NSLOT = 4
SPLIT = 128


def _row_copies(emb_hbm, idx_v, buf_v, r, slot, sem):
    return (
        pltpu.make_async_copy(
            emb_hbm.at[idx_v.at[r, pl.ds(0, SPLIT)]],
            buf_v.at[slot, pl.ds(0, SPLIT)],
            sem,
        ),
        pltpu.make_async_copy(
            emb_hbm.at[idx_v.at[r, pl.ds(SPLIT, HIST - SPLIT)]],
            buf_v.at[slot, pl.ds(SPLIT, HIST - SPLIT)],
            sem,
        ),
    )


def _pool_body(x_hbm, emb_hbm, out_hbm, idx_v, buf_v, acc_v,
               sem0, sem1, sem2, sem3):
    sems = (sem0, sem1, sem2, sem3)
    wid = lax.axis_index("s") * NC + lax.axis_index("c")
    base = wid * ROWS_PER_W

    pltpu.sync_copy(x_hbm.at[pl.ds(base, ROWS_PER_W)], idx_v)

    def issue(r, slot):
        for c in _row_copies(emb_hbm, idx_v, buf_v, r, slot, sems[slot]):
            c.start()

    def drain(r, slot):
        for c in _row_copies(emb_hbm, idx_v, buf_v, r, slot, sems[slot]):
            c.wait()

    def accumulate(slot):
        accs = [buf_v[slot, j] for j in range(8)]
        for j in range(8, HIST, 8):
            for k in range(8):
                accs[k] = accs[k] + buf_v[slot, j + k]
        accs = [accs[0] + accs[1], accs[2] + accs[3],
                accs[4] + accs[5], accs[6] + accs[7]]
        accs = [accs[0] + accs[1], accs[2] + accs[3]]
        return (accs[0] + accs[1]) * jnp.float32(1.0 / HIST)

    for k in range(NSLOT):
        issue(k, k)

    def body(i, carry):
        r = i * NSLOT
        for k in range(NSLOT):
            drain(r + k, k)
            acc_v[r + k] = accumulate(k)

            @pl.when(r + NSLOT + k < ROWS_PER_W)
            def _():
                issue(r + NSLOT + k, k)
        return carry

    lax.fori_loop(0, ROWS_PER_W // NSLOT, body, 0, unroll=False)

    pltpu.sync_copy(acc_v, out_hbm.at[pl.ds(base, ROWS_PER_W)])


@functools.partial(jax.jit, static_argnames=())
def _pool(x, emb):
    mesh = plsc.VectorSubcoreMesh(core_axis_name="c", subcore_axis_name="s")
    kern = functools.partial(
        pl.kernel,
        out_type=jax.ShapeDtypeStruct((BATCH, EMBED), jnp.float32),
        mesh=mesh,
        scratch_types=[
            pltpu.VMEM((ROWS_PER_W, HIST), jnp.int32),
            pltpu.VMEM((NSLOT, HIST, EMBED), jnp.float32),
            pltpu.VMEM((ROWS_PER_W, EMBED), jnp.float32),
            pltpu.SemaphoreType.DMA,
            pltpu.SemaphoreType.DMA,
            pltpu.SemaphoreType.DMA,
            pltpu.SemaphoreType.DMA,
        ],
        compiler_params=pltpu.CompilerParams(use_tc_tiling_on_sc=False),
    )(_pool_body)
    return kern(x, emb)


def _mlp_body(p_ref, w1t_ref, b1_ref, w2t_ref, b2_ref, o_ref):
    h = jnp.dot(p_ref[...], w1t_ref[...], preferred_element_type=jnp.float32)
    h = jnp.maximum(h + b1_ref[...], 0.0)
    o_ref[...] = (
        jnp.dot(h, w2t_ref[...], preferred_element_type=jnp.float32)
        + b2_ref[...]
    )


def _mlp(pooled, w1t, b1r, w2t, b2r):
    return pl.pallas_call(
        _mlp_body,
        out_shape=jax.ShapeDtypeStruct((BATCH, 8), jnp.float32),
    )(pooled, w1t, b1r, w2t, b2r)


def kernel(x, emb, W1, b1, W2, b2):
    pooled = _pool(x.astype(jnp.int32), emb)
    w2p = jnp.zeros((8, EMBED), jnp.float32).at[:2].set(W2)
    b2p = jnp.zeros((8,), jnp.float32).at[:2].set(b2)
    out8 = _mlp(pooled, W1.T, b1[None, :], w2p.T, b2p[None, :])
    return out8[:, :2]
